# Initial kernel scaffold; baseline (speedup 1.0000x reference)
import jax
import jax.numpy as jnp
from jax.experimental import pallas as pl


def kernel(x, batch, W_ih, W_hh, b_ih, b_hh):
    raise NotImplementedError("write your pallas kernel here")


# speedup vs baseline: 3.4589x; 3.4589x over previous
import functools

import jax
import jax.numpy as jnp
from jax import lax
from jax.experimental import pallas as pl
from jax.experimental.pallas import tpu as pltpu

_N = 50000
_C = 256
_B = 512
_STEPS = 4
_RB = 256


def _set2set_body(offs_ref, x_ref, wih_ref, whh_ref, bih_ref, bhh_ref,
                  out_ref, h_ref, c_ref, qst_ref, r_ref):
    h_ref[...] = jnp.zeros((_B, _C), jnp.float32)
    c_ref[...] = jnp.zeros((_B, _C), jnp.float32)
    qst_ref[...] = jnp.zeros((_B, 2 * _C), jnp.float32)
    b = bih_ref[...] + bhh_ref[...]

    for _ in range(_STEPS):
        gates = (
            jnp.dot(qst_ref[...], wih_ref[...], preferred_element_type=jnp.float32)
            + jnp.dot(h_ref[...], whh_ref[...], preferred_element_type=jnp.float32)
            + b
        )
        gi = jax.nn.sigmoid(gates[:, 0 * _C:1 * _C])
        gf = jax.nn.sigmoid(gates[:, 1 * _C:2 * _C])
        gg = jnp.tanh(gates[:, 2 * _C:3 * _C])
        go = jax.nn.sigmoid(gates[:, 3 * _C:4 * _C])
        c_ref[...] = gf * c_ref[...] + gi * gg
        h_ref[...] = go * jnp.tanh(c_ref[...])

        def seg_body(s, _):
            start = offs_ref[s]
            end = offs_ref[s + 1]
            base = (start // _RB) * _RB
            nb = (end - base + _RB - 1) // _RB
            q_s = h_ref[pl.ds(s, 1), :]

            def pass1(ib, m):
                r0 = base + ib * _RB
                xb = x_ref[pl.ds(r0, _RB), :]
                rows = r0 + lax.broadcasted_iota(jnp.int32, (_RB, 1), 0)
                mask = (rows >= start) & (rows < end)
                e = jnp.sum(xb * q_s, axis=1, keepdims=True)
                e = jnp.where(mask, e, -jnp.inf)
                return jnp.maximum(m, jnp.max(e))

            m = lax.fori_loop(0, nb, pass1, jnp.float32(-jnp.inf))

            def pass2(ib, carry):
                d, racc = carry
                r0 = base + ib * _RB
                xb = x_ref[pl.ds(r0, _RB), :]
                rows = r0 + lax.broadcasted_iota(jnp.int32, (_RB, 1), 0)
                mask = (rows >= start) & (rows < end)
                e = jnp.sum(xb * q_s, axis=1, keepdims=True)
                ex = jnp.where(mask, jnp.exp(e - m), 0.0)
                d = d + jnp.sum(ex)
                racc = racc + jnp.sum(ex * xb, axis=0, keepdims=True)
                return d, racc

            d, racc = lax.fori_loop(
                0, nb, pass2,
                (jnp.float32(0.0), jnp.zeros((1, _C), jnp.float32)))
            r_ref[pl.ds(s, 1), :] = racc / (d + 1e-16)
            return 0

        lax.fori_loop(0, _B, seg_body, 0)
        qst_ref[:, :_C] = h_ref[...]
        qst_ref[:, _C:] = r_ref[...]

    out_ref[...] = qst_ref[...]


@jax.jit
def kernel(x, batch, W_ih, W_hh, b_ih, b_hh):
    n_pad = ((_N + _RB - 1) // _RB) * _RB
    x_pad = jnp.pad(x, ((0, n_pad - _N), (0, 0)))
    offs = jnp.searchsorted(batch, jnp.arange(_B + 1, dtype=jnp.int32),
                            side="left").astype(jnp.int32)
    wih_t = W_ih.T
    whh_t = W_hh.T

    return pl.pallas_call(
        _set2set_body,
        out_shape=jax.ShapeDtypeStruct((_B, 2 * _C), jnp.float32),
        in_specs=[
            pl.BlockSpec(memory_space=pltpu.SMEM),
            pl.BlockSpec(memory_space=pltpu.VMEM),
            pl.BlockSpec(memory_space=pltpu.VMEM),
            pl.BlockSpec(memory_space=pltpu.VMEM),
            pl.BlockSpec(memory_space=pltpu.VMEM),
            pl.BlockSpec(memory_space=pltpu.VMEM),
        ],
        out_specs=pl.BlockSpec(memory_space=pltpu.VMEM),
        scratch_shapes=[
            pltpu.VMEM((_B, _C), jnp.float32),
            pltpu.VMEM((_B, _C), jnp.float32),
            pltpu.VMEM((_B, 2 * _C), jnp.float32),
            pltpu.VMEM((_B, _C), jnp.float32),
        ],
        compiler_params=pltpu.CompilerParams(
            vmem_limit_bytes=120 * 1024 * 1024,
        ),
    )(offs, x_pad, wih_t, whh_t, b_ih[None, :], b_hh[None, :])


# speedup vs baseline: 3.8729x; 1.1197x over previous
import functools

import jax
import jax.numpy as jnp
from jax import lax
from jax.experimental import pallas as pl
from jax.experimental.pallas import tpu as pltpu

_N = 50000
_C = 256
_B = 512
_STEPS = 4
_RB = 128


def _set2set_body(offs_ref, x_ref, wih_ref, whh_ref, bih_ref, bhh_ref,
                  out_ref, h_ref, c_ref, qst_ref, r_ref):
    h_ref[...] = jnp.zeros((_B, _C), jnp.float32)
    c_ref[...] = jnp.zeros((_B, _C), jnp.float32)
    qst_ref[...] = jnp.zeros((_B, 2 * _C), jnp.float32)
    b = bih_ref[...] + bhh_ref[...]

    for _ in range(_STEPS):
        gates = (
            jnp.dot(qst_ref[...], wih_ref[...], preferred_element_type=jnp.float32)
            + jnp.dot(h_ref[...], whh_ref[...], preferred_element_type=jnp.float32)
            + b
        )
        gi = jax.nn.sigmoid(gates[:, 0 * _C:1 * _C])
        gf = jax.nn.sigmoid(gates[:, 1 * _C:2 * _C])
        gg = jnp.tanh(gates[:, 2 * _C:3 * _C])
        go = jax.nn.sigmoid(gates[:, 3 * _C:4 * _C])
        c_ref[...] = gf * c_ref[...] + gi * gg
        h_ref[...] = go * jnp.tanh(c_ref[...])

        def seg_body(s, _):
            start = offs_ref[s]
            end = offs_ref[s + 1]
            base = (start // _RB) * _RB
            nb = (end - base + _RB - 1) // _RB
            q_s = h_ref[pl.ds(s, 1), :]

            def blk(ib, carry):
                m, d, racc = carry
                r0 = base + ib * _RB
                xb = x_ref[pl.ds(r0, _RB), :]
                rows = r0 + lax.broadcasted_iota(jnp.int32, (_RB, 1), 0)
                mask = (rows >= start) & (rows < end)
                e = jnp.sum(xb * q_s, axis=1, keepdims=True)
                e = jnp.where(mask, e, -jnp.inf)
                m_new = jnp.maximum(m, jnp.max(e))
                scale = jnp.exp(m - m_new)
                ex = jnp.where(mask, jnp.exp(e - m_new), 0.0)
                d = d * scale + jnp.sum(ex)
                racc = racc * scale + jnp.sum(ex * xb, axis=0, keepdims=True)
                return m_new, d, racc

            _, d, racc = lax.fori_loop(
                0, nb, blk,
                (jnp.float32(-jnp.inf), jnp.float32(0.0),
                 jnp.zeros((1, _C), jnp.float32)))
            r_ref[pl.ds(s, 1), :] = racc / (d + 1e-16)
            return 0

        lax.fori_loop(0, _B, seg_body, 0)
        qst_ref[:, :_C] = h_ref[...]
        qst_ref[:, _C:] = r_ref[...]

    out_ref[...] = qst_ref[...]


@jax.jit
def kernel(x, batch, W_ih, W_hh, b_ih, b_hh):
    n_pad = ((_N + _RB - 1) // _RB) * _RB
    x_pad = jnp.pad(x, ((0, n_pad - _N), (0, 0)))
    offs = jnp.searchsorted(batch, jnp.arange(_B + 1, dtype=jnp.int32),
                            side="left").astype(jnp.int32)
    wih_t = W_ih.T
    whh_t = W_hh.T

    return pl.pallas_call(
        _set2set_body,
        out_shape=jax.ShapeDtypeStruct((_B, 2 * _C), jnp.float32),
        in_specs=[
            pl.BlockSpec(memory_space=pltpu.SMEM),
            pl.BlockSpec(memory_space=pltpu.VMEM),
            pl.BlockSpec(memory_space=pltpu.VMEM),
            pl.BlockSpec(memory_space=pltpu.VMEM),
            pl.BlockSpec(memory_space=pltpu.VMEM),
            pl.BlockSpec(memory_space=pltpu.VMEM),
        ],
        out_specs=pl.BlockSpec(memory_space=pltpu.VMEM),
        scratch_shapes=[
            pltpu.VMEM((_B, _C), jnp.float32),
            pltpu.VMEM((_B, _C), jnp.float32),
            pltpu.VMEM((_B, 2 * _C), jnp.float32),
            pltpu.VMEM((_B, _C), jnp.float32),
        ],
        compiler_params=pltpu.CompilerParams(
            vmem_limit_bytes=120 * 1024 * 1024,
        ),
    )(offs, x_pad, wih_t, whh_t, b_ih[None, :], b_hh[None, :])
